# baseline (speedup 1.0000x reference)
import jax
import jax.numpy as jnp
from jax.experimental import pallas as pl

B, N, J, R = 4, 100000, 24, 128
BBOX_EXTEND = 2.2


def _batch_rodrigues(rot_vecs):
    eps = 1e-8
    angle = jnp.linalg.norm(rot_vecs + eps, axis=1, keepdims=True)
    rot_dir = rot_vecs / angle
    cos = jnp.cos(angle)[..., None]
    sin = jnp.sin(angle)[..., None]
    rx, ry, rz = rot_dir[:, 0], rot_dir[:, 1], rot_dir[:, 2]
    zeros = jnp.zeros_like(rx)
    K = jnp.stack([zeros, -rz, ry, rz, zeros, -rx, -ry, rx, zeros], axis=1).reshape(-1, 3, 3)
    ident = jnp.eye(3, dtype=rot_vecs.dtype)[None]
    return ident + sin * K + (1.0 - cos) * jnp.matmul(K, K)


def _get_tfs(poses, kintree, rest_joints, cano_tfs_inv):
    kt = jnp.asarray(kintree)
    batch_size = poses.shape[0]
    jnum = rest_joints.shape[0]
    rot_mats = _batch_rodrigues(poses.reshape(-1, 3)).reshape(batch_size, jnum, 3, 3)
    rel_joints = rest_joints.at[1:].add(-rest_joints[kt[1:]])
    rel_joints = jnp.broadcast_to(rel_joints[None], (batch_size, jnum, 3))
    top = jnp.concatenate([rot_mats, rel_joints[..., None]], axis=-1)
    bot = jnp.zeros((batch_size, jnum, 1, 4), dtype=poses.dtype)
    tm = jnp.concatenate([top, bot], axis=-2)
    tm = tm.at[:, :, 3, 3].set(1.0)
    transforms_init = jnp.zeros((batch_size, jnum, 4, 4), dtype=tm.dtype).at[:, 0].set(tm[:, 0])

    def chain_body(i, T):
        return T.at[:, i].set(jnp.matmul(T[:, kt[i]], tm[:, i]))

    transforms = jax.lax.fori_loop(1, jnum, chain_body, transforms_init)
    jh = jnp.pad(jnp.broadcast_to(rest_joints[None, ..., None], (batch_size, jnum, 3, 1)), ((0, 0), (0, 0), (0, 1), (0, 0)))
    padded = jnp.pad(jnp.matmul(transforms, jh), ((0, 0), (0, 0), (0, 0), (3, 0)))
    rel_transforms = transforms - padded
    rel_transforms = jnp.einsum('bnij,njk->bnik', rel_transforms, cano_tfs_inv)
    return rel_transforms


def _grid_sample(grid, pts):
    C, D, H, W = grid.shape

    def unnorm(c, size):
        return jnp.clip(((c + 1.0) * size - 1.0) / 2.0, 0.0, size - 1.0)

    ix = unnorm(pts[:, 0], W)
    iy = unnorm(pts[:, 1], H)
    iz = unnorm(pts[:, 2], D)
    x0 = jnp.floor(ix); y0 = jnp.floor(iy); z0 = jnp.floor(iz)
    wx1 = ix - x0; wy1 = iy - y0; wz1 = iz - z0
    wx0 = 1.0 - wx1; wy0 = 1.0 - wy1; wz0 = 1.0 - wz1
    x0i = jnp.clip(x0, 0, W - 1).astype(jnp.int32)
    x1i = jnp.clip(x0 + 1, 0, W - 1).astype(jnp.int32)
    y0i = jnp.clip(y0, 0, H - 1).astype(jnp.int32)
    y1i = jnp.clip(y0 + 1, 0, H - 1).astype(jnp.int32)
    z0i = jnp.clip(z0, 0, D - 1).astype(jnp.int32)
    z1i = jnp.clip(z0 + 1, 0, D - 1).astype(jnp.int32)
    out = jnp.zeros((C, pts.shape[0]), dtype=grid.dtype)
    for zi, wz in ((z0i, wz0), (z1i, wz1)):
        for yi, wy in ((y0i, wy0), (y1i, wy1)):
            for xi, wx in ((x0i, wx0), (x1i, wx1)):
                out = out + grid[:, zi, yi, xi] * (wz * wy * wx)[None]
    return out.T


def _pose_kernel(w_ref, t_ref, v_ref, o_ref):
    wt = jnp.dot(w_ref[0], t_ref[0], preferred_element_type=jnp.float32)
    v = v_ref[0]
    x, y, z = v[:, 0:1], v[:, 1:2], v[:, 2:3]
    r0 = wt[:, 0:1] * x + wt[:, 1:2] * y + wt[:, 2:3] * z + wt[:, 3:4]
    r1 = wt[:, 4:5] * x + wt[:, 5:6] * y + wt[:, 6:7] * z + wt[:, 7:8]
    r2 = wt[:, 8:9] * x + wt[:, 9:10] * y + wt[:, 10:11] * z + wt[:, 11:12]
    o_ref[0] = jnp.concatenate([r0, r1, r2, jnp.zeros_like(r0)], axis=1)


def kernel(verts, poses, weight_grid, rest_joints, cano_tfs_inv, kintree):
    rel_transforms = _get_tfs(poses, kintree, rest_joints, cano_tfs_inv)
    v = verts / BBOX_EXTEND * 2.0
    w = _grid_sample(weight_grid, v.reshape(B * N, 3)).reshape(B, N, J)
    t = rel_transforms.reshape(B, J, 16)
    vp = jnp.concatenate([verts, jnp.zeros((B, N, 1), jnp.float32)], axis=-1)

    blk = 2000
    out = pl.pallas_call(
        _pose_kernel,
        grid=(B, N // blk),
        in_specs=[
            pl.BlockSpec((1, blk, J), lambda b, i: (b, i, 0)),
            pl.BlockSpec((1, J, 16), lambda b, i: (b, 0, 0)),
            pl.BlockSpec((1, blk, 4), lambda b, i: (b, i, 0)),
        ],
        out_specs=pl.BlockSpec((1, blk, 4), lambda b, i: (b, i, 0)),
        out_shape=jax.ShapeDtypeStruct((B, N, 4), jnp.float32),
    )(w, t, vp)
    return out[..., :3]


# speedup vs baseline: 2.3280x; 2.3280x over previous
import functools

import jax
import jax.numpy as jnp
from jax import lax
from jax.experimental import pallas as pl
from jax.experimental.pallas import tpu as pltpu
from jax.experimental.pallas import tpu_sc as plsc

B, N, J, R = 4, 100000, 24, 128
BBOX_EXTEND = 2.2
GRID_SCALE = 2.0 * (R / 2.0) / BBOX_EXTEND

NB = 100352
NP = B * NB
NW = 32
PPW = NP // NW
CH = 256
NCHUNK = PPW // CH
NG = CH // 16
NV = R * R * R

_mesh = plsc.VectorSubcoreMesh(core_axis_name="c", subcore_axis_name="s")
_sc_params = pltpu.CompilerParams(
    use_tc_tiling_on_sc=False, needs_layout_passes=False
)


@functools.partial(
    pl.kernel,
    out_type=jax.ShapeDtypeStruct((J, NP), jnp.float32),
    mesh=_mesh,
    scratch_types=[
        pltpu.VMEM((NG, 128), jnp.int32),
        pltpu.VMEM((NG, 128), jnp.float32),
        pltpu.VMEM((3, CH), jnp.float32),
        pltpu.VMEM((CH * 8, J), jnp.float32),
        pltpu.VMEM((J, CH), jnp.float32),
        pltpu.SemaphoreType.DMA,
    ],
    compiler_params=_sc_params,
)
def _sc_weights(grid_hbm, vxyz_hbm, out_hbm, idx_v, w_v, v_v, rows_v, w24_v, sem):
    wid = lax.axis_index("s") * 2 + lax.axis_index("c")
    base0 = wid * PPW
    lane = lax.iota(jnp.int32, 16)

    def chunk_body(ci, carry):
        base = base0 + ci * CH
        pltpu.sync_copy(vxyz_hbm.at[0, pl.ds(base, CH)], v_v.at[0])
        pltpu.sync_copy(vxyz_hbm.at[1, pl.ds(base, CH)], v_v.at[1])
        pltpu.sync_copy(vxyz_hbm.at[2, pl.ds(base, CH)], v_v.at[2])

        def grp_idx(g, c2):
            sl = pl.ds(g * 16, 16)
            px = v_v[0, sl]
            py = v_v[1, sl]
            pz = v_v[2, sl]
            ix = jnp.clip(px * GRID_SCALE + (R - 1) / 2.0, 0.0, R - 1.0)
            iy = jnp.clip(py * GRID_SCALE + (R - 1) / 2.0, 0.0, R - 1.0)
            iz = jnp.clip(pz * GRID_SCALE + (R - 1) / 2.0, 0.0, R - 1.0)
            xi = jnp.minimum(ix.astype(jnp.int32), R - 2)
            yi = jnp.minimum(iy.astype(jnp.int32), R - 2)
            zi = jnp.minimum(iz.astype(jnp.int32), R - 2)
            fx = ix - xi.astype(jnp.float32)
            fy = iy - yi.astype(jnp.float32)
            fz = iz - zi.astype(jnp.float32)
            vb = zi * (R * R) + yi * R + xi
            wx = (1.0 - fx, fx)
            wy = (1.0 - fy, fy)
            wz = (1.0 - fz, fz)
            for dz in (0, 1):
                for dy in (0, 1):
                    for dx in (0, 1):
                        c = dz * 4 + dy * 2 + dx
                        idx_v[g, pl.ds(c * 16, 16)] = vb + (dz * R * R + dy * R + dx)
                        w_v[g, pl.ds(c * 16, 16)] = wz[dz] * wy[dy] * wx[dx]
            return c2

        lax.fori_loop(0, NG, grp_idx, 0)

        def fire(g, c2):
            pltpu.async_copy(
                grid_hbm.at[idx_v.at[g]],
                rows_v.at[pl.ds(g * 128, 128)],
                sem,
            )
            return c2

        lax.fori_loop(0, NG, fire, 0)
        pltpu.make_async_copy(grid_hbm.at[pl.ds(0, CH * 8)], rows_v, sem).wait()

        def grp_combine(g, c2):
            sl = pl.ds(g * 16, 16)
            rvecs = []
            wvecs = []
            for c in range(8):
                rvecs.append(g * 128 + c * 16 + lane)
                wvecs.append(w_v[g, pl.ds(c * 16, 16)])
            for ch in range(J):
                cvec = jnp.full((16,), ch, jnp.int32)
                acc = wvecs[0] * plsc.load_gather(rows_v, [rvecs[0], cvec])
                for c in range(1, 8):
                    acc = acc + wvecs[c] * plsc.load_gather(rows_v, [rvecs[c], cvec])
                w24_v[ch, sl] = acc
            return c2

        lax.fori_loop(0, NG, grp_combine, 0)
        pltpu.sync_copy(w24_v, out_hbm.at[:, pl.ds(base, CH)])
        return carry

    lax.fori_loop(0, NCHUNK, chunk_body, 0)


def _pose_body(t_ref, w_ref, v_ref, o_ref):
    t = t_ref[0]
    w = w_ref[...]
    wt = lax.dot_general(t, w, (((0,), (0,)), ((), ())),
                         preferred_element_type=jnp.float32)
    x = v_ref[0:1, :]
    y = v_ref[1:2, :]
    z = v_ref[2:3, :]
    r0 = wt[0:1] * x + wt[1:2] * y + wt[2:3] * z + wt[3:4]
    r1 = wt[4:5] * x + wt[5:6] * y + wt[6:7] * z + wt[7:8]
    r2 = wt[8:9] * x + wt[9:10] * y + wt[10:11] * z + wt[11:12]
    o_ref[...] = jnp.concatenate([r0, r1, r2], axis=0)


def _batch_rodrigues(rot_vecs):
    eps = 1e-8
    angle = jnp.linalg.norm(rot_vecs + eps, axis=1, keepdims=True)
    rot_dir = rot_vecs / angle
    cos = jnp.cos(angle)[..., None]
    sin = jnp.sin(angle)[..., None]
    rx, ry, rz = rot_dir[:, 0], rot_dir[:, 1], rot_dir[:, 2]
    zeros = jnp.zeros_like(rx)
    K = jnp.stack([zeros, -rz, ry, rz, zeros, -rx, -ry, rx, zeros], axis=1).reshape(-1, 3, 3)
    ident = jnp.eye(3, dtype=rot_vecs.dtype)[None]
    return ident + sin * K + (1.0 - cos) * jnp.matmul(K, K)


def _get_tfs(poses, kintree, rest_joints, cano_tfs_inv):
    kt = jnp.asarray(kintree)
    batch_size = poses.shape[0]
    jnum = rest_joints.shape[0]
    rot_mats = _batch_rodrigues(poses.reshape(-1, 3)).reshape(batch_size, jnum, 3, 3)
    rel_joints = rest_joints.at[1:].add(-rest_joints[kt[1:]])
    rel_joints = jnp.broadcast_to(rel_joints[None], (batch_size, jnum, 3))
    top = jnp.concatenate([rot_mats, rel_joints[..., None]], axis=-1)
    bot = jnp.zeros((batch_size, jnum, 1, 4), dtype=poses.dtype)
    tm = jnp.concatenate([top, bot], axis=-2)
    tm = tm.at[:, :, 3, 3].set(1.0)
    transforms_init = jnp.zeros((batch_size, jnum, 4, 4), dtype=tm.dtype).at[:, 0].set(tm[:, 0])

    def chain_body(i, T):
        return T.at[:, i].set(jnp.matmul(T[:, kt[i]], tm[:, i]))

    transforms = jax.lax.fori_loop(1, jnum, chain_body, transforms_init)
    jh = jnp.pad(jnp.broadcast_to(rest_joints[None, ..., None], (batch_size, jnum, 3, 1)), ((0, 0), (0, 0), (0, 1), (0, 0)))
    padded = jnp.pad(jnp.matmul(transforms, jh), ((0, 0), (0, 0), (0, 0), (3, 0)))
    rel_transforms = transforms - padded
    rel_transforms = jnp.einsum('bnij,njk->bnik', rel_transforms, cano_tfs_inv)
    return rel_transforms


def kernel(verts, poses, weight_grid, rest_joints, cano_tfs_inv, kintree):
    rel_transforms = _get_tfs(poses, kintree, rest_joints, cano_tfs_inv)
    tmat = rel_transforms[:, :, :3, :].reshape(B, J, 12)

    grid_t = weight_grid.transpose(1, 2, 3, 0).reshape(NV, J)
    verts_pad = jnp.pad(verts, ((0, 0), (0, NB - N), (0, 0)))
    vxyz = verts_pad.transpose(2, 0, 1).reshape(3, NP)

    w24 = _sc_weights(grid_t, vxyz)

    blk = 2048
    nblk = NP // blk
    per_b = NB // blk
    posed = pl.pallas_call(
        _pose_body,
        grid=(nblk,),
        in_specs=[
            pl.BlockSpec((1, J, 12), lambda k: (k // per_b, 0, 0)),
            pl.BlockSpec((J, blk), lambda k: (0, k)),
            pl.BlockSpec((3, blk), lambda k: (0, k)),
        ],
        out_specs=pl.BlockSpec((3, blk), lambda k: (0, k)),
        out_shape=jax.ShapeDtypeStruct((3, NP), jnp.float32),
    )(tmat, w24, vxyz)

    out = posed.reshape(3, B, NB).transpose(1, 2, 0)[:, :N, :]
    return out
